# baseline (speedup 1.0000x reference)
import functools

import jax
import jax.numpy as jnp
from jax import lax
from jax.experimental import pallas as pl
from jax.experimental.pallas import tpu as pltpu
from jax.experimental.pallas import tpu_sc as plsc

BATCH = 16384
DIM = 64
NUM_CORES = 2
NUM_SUBCORES = 16
LANES = 16
NW = NUM_CORES * NUM_SUBCORES
BPW = BATCH // NW
CHUNK = 128
NCHUNK = BPW // CHUNK


def _mf_body(users_hbm, items_hbm, eu_hbm, ei_hbm, out_hbm,
             idx_u, idx_i, rows_u, rows_i, out_v, sem_u, sem_i):
    wid = lax.axis_index("s") * NUM_CORES + lax.axis_index("c")
    ibase = wid * NCHUNK

    pltpu.sync_copy(users_hbm.at[pl.ds(ibase, NCHUNK)], idx_u)
    pltpu.sync_copy(items_hbm.at[pl.ds(ibase, NCHUNK)], idx_i)

    copies = []
    for k in range(NCHUNK):
        cu = pltpu.async_copy(eu_hbm.at[idx_u.at[k]],
                              rows_u.at[pl.ds(k * CHUNK, CHUNK)], sem_u)
        ci = pltpu.async_copy(ei_hbm.at[idx_i.at[k]],
                              rows_i.at[pl.ds(k * CHUNK, CHUNK)], sem_i)
        copies.append((cu, ci))

    lane_ids = lax.iota(jnp.int32, LANES)

    def block_body(b, carry):
        out_acc = jnp.zeros((LANES,), jnp.float32)
        for rr in range(LANES):
            r = b * LANES + rr
            acc = rows_u[r, pl.ds(0, LANES)] * rows_i[r, pl.ds(0, LANES)]
            for q in range(1, DIM // LANES):
                acc = acc + (rows_u[r, pl.ds(q * LANES, LANES)] *
                             rows_i[r, pl.ds(q * LANES, LANES)])
            out_acc = jnp.where(lane_ids == rr, jnp.sum(acc), out_acc)
        out_v[pl.ds(b * LANES, LANES)] = 1.0 / (1.0 + jnp.exp(-out_acc))
        return carry

    blocks_per_chunk = CHUNK // LANES
    for k in range(NCHUNK):
        copies[k][0].wait()
        copies[k][1].wait()
        lax.fori_loop(k * blocks_per_chunk, (k + 1) * blocks_per_chunk,
                      block_body, 0)

    pltpu.sync_copy(out_v, out_hbm.at[pl.ds(wid * BPW, BPW)])


_mf = functools.partial(
    pl.kernel,
    mesh=plsc.VectorSubcoreMesh(core_axis_name="c", subcore_axis_name="s"),
    compiler_params=pltpu.CompilerParams(needs_layout_passes=False,
                                         use_tc_tiling_on_sc=False),
    out_type=jax.ShapeDtypeStruct((BATCH,), jnp.float32),
    scratch_types=[
        pltpu.VMEM((NCHUNK, CHUNK), jnp.int32),
        pltpu.VMEM((NCHUNK, CHUNK), jnp.int32),
        pltpu.VMEM((BPW, DIM), jnp.float32),
        pltpu.VMEM((BPW, DIM), jnp.float32),
        pltpu.VMEM((BPW,), jnp.float32),
        pltpu.SemaphoreType.DMA,
        pltpu.SemaphoreType.DMA,
    ],
)(_mf_body)


def kernel(users, items, embedding_user, embedding_item):
    users2 = users.astype(jnp.int32).reshape(NW * NCHUNK, CHUNK)
    items2 = items.astype(jnp.int32).reshape(NW * NCHUNK, CHUNK)
    return _mf(users2, items2, embedding_user, embedding_item)
